# baseline (speedup 1.0000x reference)
import functools

import jax
import jax.numpy as jnp
from jax import lax
from jax.experimental import pallas as pl
from jax.experimental.pallas import tpu as pltpu
from jax.experimental.pallas import tpu_sc as plsc

VOCAB = 1000000
EMB = 64
HID = 100
B = 1024
T = 200
GP = 128
NG = 4 * GP

NW = 32
BPW = B // NW
TU = T // 2
TUP = 104
NBC = BPW // 2


def _gather_sc(emb, idx_e, idx_o):
    mesh = plsc.VectorSubcoreMesh(core_axis_name="c", subcore_axis_name="s")

    @functools.partial(
        pl.kernel,
        mesh=mesh,
        out_type=(
            jax.ShapeDtypeStruct((B, TUP, EMB), jnp.float32),
            jax.ShapeDtypeStruct((B, TUP, EMB), jnp.float32),
        ),
        scratch_types=[
            pltpu.VMEM((BPW, TUP), jnp.int32),
            pltpu.VMEM((BPW, TUP), jnp.int32),
            pltpu.VMEM((TUP, EMB), jnp.float32),
            pltpu.VMEM((TUP, EMB), jnp.float32),
            pltpu.SemaphoreType.DMA,
        ],
        compiler_params=pltpu.CompilerParams(use_tc_tiling_on_sc=False),
    )
    def k(emb_hbm, idxe_hbm, idxo_hbm, oute_hbm, outo_hbm,
          idx_ve, idx_vo, buf_e, buf_o, sem):
        wid = lax.axis_index("s") * 2 + lax.axis_index("c")
        pltpu.sync_copy(idxe_hbm.at[wid], idx_ve)
        pltpu.sync_copy(idxo_hbm.at[wid], idx_vo)

        def body(bl, carry):
            bg = wid * BPW + bl
            cp1 = pltpu.async_copy(emb_hbm.at[idx_ve.at[bl]], buf_e, sem)
            cp2 = pltpu.async_copy(emb_hbm.at[idx_vo.at[bl]], buf_o, sem)
            cp1.wait()
            cp2.wait()
            pltpu.sync_copy(buf_e, oute_hbm.at[bg])
            pltpu.sync_copy(buf_o, outo_hbm.at[bg])
            return carry

        lax.fori_loop(0, BPW, body, 0)

    return k(emb, idx_e, idx_o)


TS = 4
NT = T // TS


def _lstm_body(ee_ref, eo_ref, wih_ref, whh_ref, b_ref, fcw_ref, fcb_ref,
               out_ref, h_ref, c_ref):
    tb = pl.program_id(0)

    @pl.when(tb == 0)
    def _init():
        h_ref[...] = jnp.zeros_like(h_ref)
        c_ref[...] = jnp.zeros_like(c_ref)

    h = h_ref[...]
    c = c_ref[...]
    for k in range(TS):
        src = ee_ref if k % 2 == 0 else eo_ref
        col = (k // 2) * EMB
        e_t = src[:, col:col + EMB]
        gates = (jnp.dot(e_t, wih_ref[...], preferred_element_type=jnp.float32)
                 + jnp.dot(h, whh_ref[...], preferred_element_type=jnp.float32)
                 + b_ref[...])
        i = jax.nn.sigmoid(gates[:, 0:GP])
        f = jax.nn.sigmoid(gates[:, GP:2 * GP])
        g = jnp.tanh(gates[:, 2 * GP:3 * GP])
        o = jax.nn.sigmoid(gates[:, 3 * GP:4 * GP])
        c = f * c + i * g
        h = o * jnp.tanh(c)
    h_ref[...] = h
    c_ref[...] = c

    @pl.when(tb == NT - 1)
    def _head():
        out_ref[...] = jax.nn.sigmoid(
            jnp.sum(h * fcw_ref[...], axis=1, keepdims=True) + fcb_ref[...])


def _lstm_tc(e_e, e_o, wih_p, whh_p, b_p, fcw_p, fcb_p):
    return pl.pallas_call(
        _lstm_body,
        grid=(NT,),
        in_specs=[
            pl.BlockSpec((B, 2 * EMB), lambda t: (0, t)),
            pl.BlockSpec((B, 2 * EMB), lambda t: (0, t)),
            pl.BlockSpec((EMB, NG), lambda t: (0, 0)),
            pl.BlockSpec((GP, NG), lambda t: (0, 0)),
            pl.BlockSpec((1, NG), lambda t: (0, 0)),
            pl.BlockSpec((1, GP), lambda t: (0, 0)),
            pl.BlockSpec((1, 1), lambda t: (0, 0)),
        ],
        out_specs=pl.BlockSpec((B, 1), lambda t: (0, 0)),
        out_shape=jax.ShapeDtypeStruct((B, 1), jnp.float32),
        scratch_shapes=[
            pltpu.VMEM((B, GP), jnp.float32),
            pltpu.VMEM((B, GP), jnp.float32),
        ],
    )(e_e, e_o, wih_p, whh_p, b_p, fcw_p, fcb_p)


def kernel(x, emb, W_ih, W_hh, b_ih, b_hh, fc_w, fc_b):
    xi = x.astype(jnp.int32).reshape(B, TU, 2)
    xi = jnp.pad(jnp.swapaxes(xi, 1, 2), ((0, 0), (0, 0), (0, TUP - TU)))
    idx_e = xi[:, 0, :].reshape(NW, BPW, TUP)
    idx_o = xi[:, 1, :].reshape(NW, BPW, TUP)
    out_e, out_o = _gather_sc(emb, idx_e, idx_o)
    e_e = out_e.reshape(B, TUP * EMB)
    e_o = out_o.reshape(B, TUP * EMB)

    w_ih4 = W_ih.reshape(4, HID, EMB)
    wih_p = jnp.zeros((4, GP, EMB), jnp.float32).at[:, :HID, :].set(w_ih4)
    wih_p = wih_p.reshape(NG, EMB).T
    w_hh4 = W_hh.reshape(4, HID, HID)
    whh_p = jnp.zeros((4, GP, GP), jnp.float32).at[:, :HID, :HID].set(w_hh4)
    whh_p = whh_p.reshape(NG, GP).T
    b4 = (b_ih + b_hh).reshape(4, HID)
    b_p = jnp.zeros((4, GP), jnp.float32).at[:, :HID].set(b4).reshape(1, NG)
    fcw_p = jnp.zeros((1, GP), jnp.float32).at[:, :HID].set(fc_w)
    fcb_p = fc_b.reshape(1, 1)

    out = _lstm_tc(e_e, e_o, wih_p, whh_p, b_p, fcw_p, fcb_p)
    return out.reshape(B)


# speedup vs baseline: 1.3944x; 1.3944x over previous
import functools

import jax
import jax.numpy as jnp
from jax import lax
from jax.experimental import pallas as pl
from jax.experimental.pallas import tpu as pltpu
from jax.experimental.pallas import tpu_sc as plsc

VOCAB = 1000000
EMB = 64
HID = 100
B = 1024
T = 200
GP = 128
NG = 4 * GP

NW = 32
TOT = B * T
PER_W = TOT // NW
CHUNK = 128
NCH = PER_W // CHUNK


def _gather_sc(emb, idx3):
    mesh = plsc.VectorSubcoreMesh(core_axis_name="c", subcore_axis_name="s")

    @functools.partial(
        pl.kernel,
        mesh=mesh,
        out_type=jax.ShapeDtypeStruct((TOT, EMB), jnp.float32),
        scratch_types=[
            pltpu.VMEM((NCH, CHUNK), jnp.int32),
            pltpu.VMEM((CHUNK, EMB), jnp.float32),
            pltpu.VMEM((CHUNK, EMB), jnp.float32),
            pltpu.SemaphoreType.DMA,
            pltpu.SemaphoreType.DMA,
        ],
        compiler_params=pltpu.CompilerParams(use_tc_tiling_on_sc=False),
    )
    def k(emb_hbm, idx_hbm, out_hbm, idx_v, rows_a, rows_b, sem_a, sem_b):
        wid = lax.axis_index("s") * 2 + lax.axis_index("c")
        pltpu.sync_copy(idx_hbm.at[wid], idx_v)
        base = wid * PER_W

        pltpu.async_copy(emb_hbm.at[idx_v.at[0]], rows_a, sem_a)

        def body(kk, carry):
            j = 2 * kk
            pltpu.async_copy(emb_hbm.at[idx_v.at[j + 1]], rows_b, sem_b)
            pltpu.make_async_copy(emb_hbm.at[idx_v.at[j]], rows_a, sem_a).wait()
            pltpu.sync_copy(rows_a, out_hbm.at[pl.ds(base + j * CHUNK, CHUNK)])

            @pl.when(j + 2 < NCH)
            def _():
                pltpu.async_copy(emb_hbm.at[idx_v.at[j + 2]], rows_a, sem_a)

            pltpu.make_async_copy(
                emb_hbm.at[idx_v.at[j + 1]], rows_b, sem_b).wait()
            pltpu.sync_copy(
                rows_b, out_hbm.at[pl.ds(base + (j + 1) * CHUNK, CHUNK)])
            return carry

        lax.fori_loop(0, NCH // 2, body, 0)

    return k(emb, idx3)


TS = 4
NT = T // TS


def _lstm_body(e_ref, wih_ref, whh_ref, b_ref, fcw_ref, fcb_ref,
               out_ref, h_ref, c_ref):
    tb = pl.program_id(0)

    @pl.when(tb == 0)
    def _init():
        h_ref[...] = jnp.zeros_like(h_ref)
        c_ref[...] = jnp.zeros_like(c_ref)

    h = h_ref[...]
    c = c_ref[...]
    for k in range(TS):
        e_t = e_ref[:, k * EMB:(k + 1) * EMB].astype(jnp.bfloat16)
        gates = (jnp.dot(e_t, wih_ref[...], preferred_element_type=jnp.float32)
                 + jnp.dot(h.astype(jnp.bfloat16), whh_ref[...],
                           preferred_element_type=jnp.float32)
                 + b_ref[...])
        i = jax.nn.sigmoid(gates[:, 0:GP])
        f = jax.nn.sigmoid(gates[:, GP:2 * GP])
        g = jnp.tanh(gates[:, 2 * GP:3 * GP])
        o = jax.nn.sigmoid(gates[:, 3 * GP:4 * GP])
        c = f * c + i * g
        h = o * jnp.tanh(c)
    h_ref[...] = h
    c_ref[...] = c

    @pl.when(tb == NT - 1)
    def _head():
        out_ref[...] = jax.nn.sigmoid(
            jnp.sum(h * fcw_ref[...], axis=1, keepdims=True) + fcb_ref[...])


def _lstm_tc(e_bte, wih_p, whh_p, b_p, fcw_p, fcb_p):
    return pl.pallas_call(
        _lstm_body,
        grid=(NT,),
        in_specs=[
            pl.BlockSpec((B, TS * EMB), lambda t: (0, t)),
            pl.BlockSpec((EMB, NG), lambda t: (0, 0)),
            pl.BlockSpec((GP, NG), lambda t: (0, 0)),
            pl.BlockSpec((1, NG), lambda t: (0, 0)),
            pl.BlockSpec((1, GP), lambda t: (0, 0)),
            pl.BlockSpec((1, 1), lambda t: (0, 0)),
        ],
        out_specs=pl.BlockSpec((B, 1), lambda t: (0, 0)),
        out_shape=jax.ShapeDtypeStruct((B, 1), jnp.float32),
        scratch_shapes=[
            pltpu.VMEM((B, GP), jnp.float32),
            pltpu.VMEM((B, GP), jnp.float32),
        ],
    )(e_bte, wih_p, whh_p, b_p, fcw_p, fcb_p)


def kernel(x, emb, W_ih, W_hh, b_ih, b_hh, fc_w, fc_b):
    idx3 = x.astype(jnp.int32).reshape(NW, NCH, CHUNK)
    e = _gather_sc(emb, idx3).reshape(B, T * EMB)

    w_ih4 = W_ih.reshape(4, HID, EMB)
    wih_p = jnp.zeros((4, GP, EMB), jnp.float32).at[:, :HID, :].set(w_ih4)
    wih_p = wih_p.reshape(NG, EMB).T.astype(jnp.bfloat16)
    w_hh4 = W_hh.reshape(4, HID, HID)
    whh_p = jnp.zeros((4, GP, GP), jnp.float32).at[:, :HID, :HID].set(w_hh4)
    whh_p = whh_p.reshape(NG, GP).T.astype(jnp.bfloat16)
    b4 = (b_ih + b_hh).reshape(4, HID)
    b_p = jnp.zeros((4, GP), jnp.float32).at[:, :HID].set(b4).reshape(1, NG)
    fcw_p = jnp.zeros((1, GP), jnp.float32).at[:, :HID].set(fc_w)
    fcb_p = fc_b.reshape(1, 1)

    out = _lstm_tc(e, wih_p, whh_p, b_p, fcw_p, fcb_p)
    return out.reshape(B)
